# baseline (speedup 1.0000x reference)
import functools

import jax
import jax.numpy as jnp
from jax import lax
from jax.experimental import pallas as pl
from jax.experimental.pallas import tpu as pltpu
from jax.experimental.pallas import tpu_sc as plsc

N = 10000
NP = 10240
NPU = 10240
E = 160000
D_HID = 256
BN_EPS = 1e-5

NC, NS = 2, 16
H = 5000
ACC_ROWS = 5120
DUMMY = 5100
CH_E = 128
CHT = 80
IBP = 16
ROWS_PER_TILE = ACC_ROWS // NS
INIT_W = 64
INIT_CH = ROWS_PER_TILE // INIT_W

LANES = 128
DEG_CH = 40
DEG_RPT = NP // NS

BR = 1000


def _sc_degree(dst_idx):
    mesh = plsc.VectorSubcoreMesh(core_axis_name="c", subcore_axis_name="s")

    @functools.partial(
        pl.kernel,
        out_type=jax.ShapeDtypeStruct((NC, NP), jnp.float32),
        mesh=mesh,
        scratch_types=[
            pltpu.VMEM((DEG_CH, LANES), jnp.int32),
            pltpu.VMEM((LANES,), jnp.float32),
            pltpu.VMEM((DEG_RPT,), jnp.float32),
            pltpu.VMEM_SHARED((NP,), jnp.float32),
        ],
    )
    def deg_kernel(dst_hbm, out_hbm, idx_v, ones_v, stage_v, acc_sh):
        c = lax.axis_index("c")
        s = lax.axis_index("s")
        for k in range(LANES // 16):
            ones_v[pl.ds(k * 16, 16)] = jnp.full((16,), 1.0, jnp.float32)
        for k in range(DEG_RPT // 16):
            stage_v[pl.ds(k * 16, 16)] = jnp.zeros((16,), jnp.float32)
        pltpu.sync_copy(dst_hbm.at[c].at[s], idx_v)
        pltpu.sync_copy(stage_v, acc_sh.at[pl.ds(s * DEG_RPT, DEG_RPT)])
        plsc.subcore_barrier()

        def body(j, carry):
            pltpu.sync_copy(ones_v, acc_sh.at[idx_v.at[j]], add=True)
            return carry

        lax.fori_loop(0, DEG_CH, body, 0)
        plsc.subcore_barrier()
        pltpu.sync_copy(acc_sh.at[pl.ds(s * DEG_RPT, DEG_RPT)], stage_v)
        pltpu.sync_copy(stage_v, out_hbm.at[c].at[pl.ds(s * DEG_RPT, DEG_RPT)])

    return deg_kernel(dst_idx)


def _sc_propagate(u, src_idx, dst_idx):
    mesh = plsc.VectorSubcoreMesh(core_axis_name="c", subcore_axis_name="s")

    @functools.partial(
        pl.kernel,
        out_type=jax.ShapeDtypeStruct((NC, ACC_ROWS, 2, 128), jnp.float32),
        mesh=mesh,
        scratch_types=[
            pltpu.VMEM((IBP, CH_E), jnp.int32),
            pltpu.VMEM((IBP, CH_E), jnp.int32),
            pltpu.VMEM((CH_E, 2, 128), jnp.float32),
            pltpu.VMEM_SHARED((ACC_ROWS, 2, 128), jnp.float32),
            pltpu.SemaphoreType.DMA,
        ],
    )
    def prop_kernel(u_hbm, src_hbm, dst_hbm, out_hbm, src_v, dst_v,
                    ebuf, acc_sh, sem0):
        c = lax.axis_index("c")
        s = lax.axis_index("s")
        row0 = s * ROWS_PER_TILE
        for k in range(INIT_CH):
            pltpu.sync_copy(u_hbm.at[pl.ds(c * H + row0 + k * INIT_W, INIT_W)],
                            ebuf.at[pl.ds(0, INIT_W)])
            pltpu.sync_copy(ebuf.at[pl.ds(0, INIT_W)],
                            acc_sh.at[pl.ds(row0 + k * INIT_W, INIT_W)])
        plsc.subcore_barrier()

        def outer(bi, carry):
            pltpu.sync_copy(src_hbm.at[s].at[pl.ds(bi * IBP, IBP)], src_v)
            pltpu.sync_copy(dst_hbm.at[c].at[s].at[pl.ds(bi * IBP, IBP)], dst_v)

            def body(j, carry2):
                pltpu.async_copy(u_hbm.at[src_v.at[j]], ebuf, sem0).wait()
                pltpu.sync_copy(ebuf, acc_sh.at[dst_v.at[j]], add=True)
                return carry2

            lax.fori_loop(0, IBP, body, 0)
            return carry

        lax.fori_loop(0, CHT // IBP, outer, 0)
        plsc.subcore_barrier()
        for k in range(INIT_CH):
            pltpu.sync_copy(acc_sh.at[pl.ds(row0 + k * INIT_W, INIT_W)],
                            ebuf.at[pl.ds(0, INIT_W)])
            pltpu.sync_copy(ebuf.at[pl.ds(0, INIT_W)],
                            out_hbm.at[c].at[pl.ds(row0 + k * INIT_W, INIT_W)])

    return prop_kernel(u.reshape(NPU, 2, 128), src_idx, dst_idx)


def _tc_dis(deg_partials):
    def body(p_ref, o_ref):
        p = p_ref[...]
        o_ref[...] = lax.rsqrt(p[0:1, :] + p[1:2, :] + 1.0)

    return pl.pallas_call(
        body, out_shape=jax.ShapeDtypeStruct((1, NP), jnp.float32)
    )(deg_partials)


def _tc_umat(h, W, dis_col):
    def body(h_ref, w_ref, d_ref, o_ref):
        acc = jnp.dot(h_ref[...], w_ref[...], preferred_element_type=jnp.float32)
        o_ref[...] = acc * d_ref[...]

    return pl.pallas_call(
        body,
        grid=(N // BR,),
        in_specs=[
            pl.BlockSpec((BR, D_HID), lambda i: (i, 0)),
            pl.BlockSpec((D_HID, D_HID), lambda i: (0, 0)),
            pl.BlockSpec((BR, 1), lambda i: (i, 0)),
        ],
        out_specs=pl.BlockSpec((BR, D_HID), lambda i: (i, 0)),
        out_shape=jax.ShapeDtypeStruct((NPU, D_HID), jnp.float32),
    )(h, W, dis_col)


def _tc_postin(a, dis_col, b):
    def body(a_ref, d_ref, b_ref, o_ref):
        o_ref[...] = a_ref[0] * d_ref[...] + b_ref[...]

    return pl.pallas_call(
        body,
        grid=(NC, H // BR),
        in_specs=[
            pl.BlockSpec((1, BR, D_HID), lambda h, i: (h, i, 0)),
            pl.BlockSpec((BR, 1), lambda h, i: (h * (H // BR) + i, 0)),
            pl.BlockSpec((1, D_HID), lambda h, i: (0, 0)),
        ],
        out_specs=pl.BlockSpec((BR, D_HID), lambda h, i: (h * (H // BR) + i, 0)),
        out_shape=jax.ShapeDtypeStruct((N, D_HID), jnp.float32),
    )(a, dis_col, b.reshape(1, D_HID))


def _tc_stats(a, dis_col, b):
    def body(a_ref, d_ref, b_ref, o_ref):
        hh = pl.program_id(0)
        i = pl.program_id(1)
        y = a_ref[0] * d_ref[...] + b_ref[...]
        ps = jnp.sum(y, axis=0, keepdims=True)
        ps2 = jnp.sum(y * y, axis=0, keepdims=True)

        @pl.when(jnp.logical_and(hh == 0, i == 0))
        def _():
            o_ref[...] = jnp.zeros_like(o_ref)

        o_ref[...] += jnp.concatenate([ps, ps2], axis=0)

    return pl.pallas_call(
        body,
        grid=(NC, H // BR),
        in_specs=[
            pl.BlockSpec((1, BR, D_HID), lambda h, i: (h, i, 0)),
            pl.BlockSpec((BR, 1), lambda h, i: (h * (H // BR) + i, 0)),
            pl.BlockSpec((1, D_HID), lambda h, i: (0, 0)),
        ],
        out_specs=pl.BlockSpec((2, D_HID), lambda h, i: (0, 0)),
        out_shape=jax.ShapeDtypeStruct((2, D_HID), jnp.float32),
    )(a, dis_col, b.reshape(1, D_HID))


def _tc_finish(a, dis_col, b, gamma, beta, stats, h_old):
    def body(a_ref, d_ref, b_ref, g_ref, be_ref, st_ref, h_ref, o_ref):
        y = a_ref[0] * d_ref[...] + b_ref[...]
        mean = st_ref[0:1, :] * (1.0 / N)
        var = st_ref[1:2, :] * (1.0 / N) - mean * mean
        z = (y - mean) * lax.rsqrt(var + BN_EPS) * g_ref[...] + be_ref[...]
        sig = 1.0 / (1.0 + jnp.exp(-z))
        o_ref[...] = h_ref[...] + z * sig

    nb = H // BR
    return pl.pallas_call(
        body,
        grid=(NC, nb),
        in_specs=[
            pl.BlockSpec((1, BR, D_HID), lambda h, i: (h, i, 0)),
            pl.BlockSpec((BR, 1), lambda h, i: (h * nb + i, 0)),
            pl.BlockSpec((1, D_HID), lambda h, i: (0, 0)),
            pl.BlockSpec((1, D_HID), lambda h, i: (0, 0)),
            pl.BlockSpec((1, D_HID), lambda h, i: (0, 0)),
            pl.BlockSpec((2, D_HID), lambda h, i: (0, 0)),
            pl.BlockSpec((BR, D_HID), lambda h, i: (h * nb + i, 0)),
        ],
        out_specs=pl.BlockSpec((BR, D_HID), lambda h, i: (h * nb + i, 0)),
        out_shape=jax.ShapeDtypeStruct((N, D_HID), jnp.float32),
    )(a, dis_col, b.reshape(1, D_HID), gamma.reshape(1, D_HID),
      beta.reshape(1, D_HID), stats, h_old)


def _tc_readout(h, W_out, b_out):
    def body(h_ref, w_ref, b_ref, o_ref):
        o_ref[...] = (
            jnp.dot(h_ref[...], w_ref[...], preferred_element_type=jnp.float32)
            + b_ref[...]
        )

    dout = W_out.shape[1]
    return pl.pallas_call(
        body,
        grid=(N // BR,),
        in_specs=[
            pl.BlockSpec((BR, D_HID), lambda i: (i, 0)),
            pl.BlockSpec((D_HID, dout), lambda i: (0, 0)),
            pl.BlockSpec((1, dout), lambda i: (0, 0)),
        ],
        out_specs=pl.BlockSpec((BR, dout), lambda i: (i, 0)),
        out_shape=jax.ShapeDtypeStruct((N, dout), jnp.float32),
    )(h, W_out, b_out.reshape(1, dout))


def kernel(x, edge_index, W_in, b_in, Ws, bs, gammas, betas, W_out, b_out):
    src = edge_index[0].astype(jnp.int32)
    dst = edge_index[1].astype(jnp.int32)

    n_pad = NC * NS * DEG_CH * LANES - E
    dst_deg = jnp.concatenate(
        [dst, jnp.full((n_pad,), NP - 1, jnp.int32)]
    ).reshape(NC, NS, DEG_CH, LANES)

    p_pad = NS * CHT * CH_E - E
    src_p = jnp.concatenate([src, jnp.zeros((p_pad,), jnp.int32)])
    src_prop = src_p.reshape(NS, CHT, CH_E)
    dst_locals = []
    for c in range(NC):
        local = jnp.where(
            jnp.logical_and(dst >= c * H, dst < (c + 1) * H), dst - c * H, DUMMY
        )
        dst_locals.append(jnp.concatenate([local, jnp.full((p_pad,), DUMMY, jnp.int32)]))
    dst_prop = jnp.stack(dst_locals).reshape(NC, NS, CHT, CH_E)

    deg_part = _sc_degree(dst_deg)
    dis_row = _tc_dis(deg_part)
    dis_col = dis_row[0, :N].reshape(N, 1)

    u = _tc_umat(x, W_in, dis_col)
    a = _sc_propagate(u, src_prop, dst_prop).reshape(NC, ACC_ROWS, D_HID)
    h = _tc_postin(a, dis_col, b_in)
    for l in range(Ws.shape[0]):
        u = _tc_umat(h, Ws[l], dis_col)
        a = _sc_propagate(u, src_prop, dst_prop).reshape(NC, ACC_ROWS, D_HID)
        st = _tc_stats(a, dis_col, bs[l])
        h = _tc_finish(a, dis_col, bs[l], gammas[l], betas[l], st, h)
    return _tc_readout(h, W_out, b_out)


# speedup vs baseline: 2.3418x; 2.3418x over previous
import functools

import jax
import jax.numpy as jnp
from jax import lax
from jax.experimental import pallas as pl
from jax.experimental.pallas import tpu as pltpu
from jax.experimental.pallas import tpu_sc as plsc

N = 10000
NP = 10240
E = 160000
D_HID = 256
DH = 128
BN_EPS = 1e-5

NC, NS = 2, 16
LANES = 128
DEG_CH = 40
PROP_CH = 80
ROWS_PER_TILE = NP // NS
INIT_CH = ROWS_PER_TILE // LANES
IB = 32

BR = 1000


def _sc_degree(dst_idx):
    mesh = plsc.VectorSubcoreMesh(core_axis_name="c", subcore_axis_name="s")

    @functools.partial(
        pl.kernel,
        out_type=jax.ShapeDtypeStruct((NC, NP), jnp.float32),
        mesh=mesh,
        scratch_types=[
            pltpu.VMEM((DEG_CH, LANES), jnp.int32),
            pltpu.VMEM((LANES,), jnp.float32),
            pltpu.VMEM((ROWS_PER_TILE,), jnp.float32),
            pltpu.VMEM_SHARED((NP,), jnp.float32),
        ],
    )
    def deg_kernel(dst_hbm, out_hbm, idx_v, ones_v, stage_v, acc_sh):
        c = lax.axis_index("c")
        s = lax.axis_index("s")
        for k in range(LANES // 16):
            ones_v[pl.ds(k * 16, 16)] = jnp.full((16,), 1.0, jnp.float32)
        for k in range(ROWS_PER_TILE // 16):
            stage_v[pl.ds(k * 16, 16)] = jnp.zeros((16,), jnp.float32)
        pltpu.sync_copy(dst_hbm.at[c].at[s], idx_v)
        pltpu.sync_copy(stage_v, acc_sh.at[pl.ds(s * ROWS_PER_TILE, ROWS_PER_TILE)])
        plsc.subcore_barrier()

        def body(j, carry):
            pltpu.sync_copy(ones_v, acc_sh.at[idx_v.at[j]], add=True)
            return carry

        lax.fori_loop(0, DEG_CH, body, 0)
        plsc.subcore_barrier()
        pltpu.sync_copy(acc_sh.at[pl.ds(s * ROWS_PER_TILE, ROWS_PER_TILE)], stage_v)
        pltpu.sync_copy(stage_v, out_hbm.at[c].at[pl.ds(s * ROWS_PER_TILE, ROWS_PER_TILE)])

    return deg_kernel(dst_idx)


def _sc_propagate(u, src_idx, dst_idx):
    mesh = plsc.VectorSubcoreMesh(core_axis_name="c", subcore_axis_name="s")

    @functools.partial(
        pl.kernel,
        out_type=jax.ShapeDtypeStruct((NC * NP, DH), jnp.float32),
        mesh=mesh,
        scratch_types=[
            pltpu.VMEM((IB, LANES), jnp.int32),
            pltpu.VMEM((IB, LANES), jnp.int32),
            pltpu.VMEM((LANES, DH), jnp.float32),
            pltpu.VMEM((LANES, DH), jnp.float32),
            pltpu.VMEM_SHARED((NP, DH), jnp.float32),
            pltpu.SemaphoreType.DMA,
            pltpu.SemaphoreType.DMA,
        ],
    )
    def prop_kernel(u_hbm, src_hbm, dst_hbm, out_hbm, src_v, dst_v,
                    ebuf0, ebuf1, acc_sh, sem0, sem1):
        c = lax.axis_index("c")
        s = lax.axis_index("s")
        row0 = s * ROWS_PER_TILE
        for k in range(INIT_CH):
            pltpu.sync_copy(u_hbm.at[pl.ds(c * NP + row0 + k * LANES, LANES)], ebuf0)
            pltpu.sync_copy(ebuf0, acc_sh.at[pl.ds(row0 + k * LANES, LANES)])
        plsc.subcore_barrier()

        def outer(bi, carry):
            pltpu.sync_copy(src_hbm.at[c].at[s].at[pl.ds(bi * IB, IB)], src_v)
            pltpu.sync_copy(dst_hbm.at[s].at[pl.ds(bi * IB, IB)], dst_v)

            def body(jj, carry2):
                j0 = 2 * jj
                cp0 = pltpu.async_copy(u_hbm.at[src_v.at[j0]], ebuf0, sem0)
                cp1 = pltpu.async_copy(u_hbm.at[src_v.at[j0 + 1]], ebuf1, sem1)
                cp0.wait()
                pltpu.sync_copy(ebuf0, acc_sh.at[dst_v.at[j0]], add=True)
                cp1.wait()
                pltpu.sync_copy(ebuf1, acc_sh.at[dst_v.at[j0 + 1]], add=True)
                return carry2

            lax.fori_loop(0, IB // 2, body, 0)
            return carry

        lax.fori_loop(0, PROP_CH // IB, outer, 0)
        plsc.subcore_barrier()
        for k in range(INIT_CH):
            pltpu.sync_copy(acc_sh.at[pl.ds(row0 + k * LANES, LANES)], ebuf0)
            pltpu.sync_copy(ebuf0, out_hbm.at[pl.ds(c * NP + row0 + k * LANES, LANES)])

    return prop_kernel(u, src_idx, dst_idx)


def _tc_dis(deg_partials):
    def body(p_ref, o_ref):
        p = p_ref[...]
        o_ref[...] = lax.rsqrt(p[0:1, :] + p[1:2, :] + 1.0)

    return pl.pallas_call(
        body, out_shape=jax.ShapeDtypeStruct((1, NP), jnp.float32)
    )(deg_partials)


def _tc_umat(h, W, dis_col):
    def body(h_ref, w_ref, d_ref, o_ref):
        acc = jnp.dot(h_ref[...], w_ref[...], preferred_element_type=jnp.float32)
        o_ref[...] = (acc * d_ref[...])[None]

    return pl.pallas_call(
        body,
        grid=(N // BR, 2),
        in_specs=[
            pl.BlockSpec((BR, D_HID), lambda i, j: (i, 0)),
            pl.BlockSpec((D_HID, DH), lambda i, j: (0, j)),
            pl.BlockSpec((BR, 1), lambda i, j: (i, 0)),
        ],
        out_specs=pl.BlockSpec((1, BR, DH), lambda i, j: (j, i, 0)),
        out_shape=jax.ShapeDtypeStruct((NC, NP, DH), jnp.float32),
    )(h, W, dis_col)


def _tc_postin_umat(a, dis_col, b, W):
    def body(a0_ref, a1_ref, d_ref, b_ref, w_ref, h_ref, u_ref):
        hh = jnp.concatenate([a0_ref[0], a1_ref[0]], axis=1) * d_ref[...] + b_ref[...]
        h_ref[...] = hh
        acc = jnp.dot(hh, w_ref[...], preferred_element_type=jnp.float32)
        u_ref[...] = (acc * d_ref[...])[None]

    return pl.pallas_call(
        body,
        grid=(N // BR, 2),
        in_specs=[
            pl.BlockSpec((1, BR, DH), lambda i, j: (0, i, 0)),
            pl.BlockSpec((1, BR, DH), lambda i, j: (1, i, 0)),
            pl.BlockSpec((BR, 1), lambda i, j: (i, 0)),
            pl.BlockSpec((1, D_HID), lambda i, j: (0, 0)),
            pl.BlockSpec((D_HID, DH), lambda i, j: (0, j)),
        ],
        out_specs=[
            pl.BlockSpec((BR, D_HID), lambda i, j: (i, 0)),
            pl.BlockSpec((1, BR, DH), lambda i, j: (j, i, 0)),
        ],
        out_shape=[
            jax.ShapeDtypeStruct((N, D_HID), jnp.float32),
            jax.ShapeDtypeStruct((NC, NP, DH), jnp.float32),
        ],
    )(a, a, dis_col, b.reshape(1, D_HID), W)


def _tc_stats(a, dis_col, b):
    def body(a0_ref, a1_ref, d_ref, b_ref, o_ref):
        i = pl.program_id(0)
        y = jnp.concatenate([a0_ref[0], a1_ref[0]], axis=1) * d_ref[...] + b_ref[...]
        ps = jnp.sum(y, axis=0, keepdims=True)
        ps2 = jnp.sum(y * y, axis=0, keepdims=True)

        @pl.when(i == 0)
        def _():
            o_ref[...] = jnp.zeros_like(o_ref)

        o_ref[...] += jnp.concatenate([ps, ps2], axis=0)

    return pl.pallas_call(
        body,
        grid=(N // BR,),
        in_specs=[
            pl.BlockSpec((1, BR, DH), lambda i: (0, i, 0)),
            pl.BlockSpec((1, BR, DH), lambda i: (1, i, 0)),
            pl.BlockSpec((BR, 1), lambda i: (i, 0)),
            pl.BlockSpec((1, D_HID), lambda i: (0, 0)),
        ],
        out_specs=pl.BlockSpec((2, D_HID), lambda i: (0, 0)),
        out_shape=jax.ShapeDtypeStruct((2, D_HID), jnp.float32),
    )(a, a, dis_col, b.reshape(1, D_HID))


def _bn_silu_res(a0, a1, d, b, g, be, st, h_old):
    y = jnp.concatenate([a0, a1], axis=1) * d + b
    mean = st[0:1, :] * (1.0 / N)
    var = st[1:2, :] * (1.0 / N) - mean * mean
    z = (y - mean) * lax.rsqrt(var + BN_EPS) * g + be
    sig = 1.0 / (1.0 + jnp.exp(-z))
    return h_old + z * sig


def _tc_finish_umat(a, dis_col, b, gamma, beta, stats, h_old, W):
    def body(a0_ref, a1_ref, d_ref, b_ref, g_ref, be_ref, st_ref, hp_ref,
             w_ref, h_ref, u_ref):
        hh = _bn_silu_res(a0_ref[0], a1_ref[0], d_ref[...], b_ref[...],
                          g_ref[...], be_ref[...], st_ref[...], hp_ref[...])
        h_ref[...] = hh
        acc = jnp.dot(hh, w_ref[...], preferred_element_type=jnp.float32)
        u_ref[...] = (acc * d_ref[...])[None]

    return pl.pallas_call(
        body,
        grid=(N // BR, 2),
        in_specs=[
            pl.BlockSpec((1, BR, DH), lambda i, j: (0, i, 0)),
            pl.BlockSpec((1, BR, DH), lambda i, j: (1, i, 0)),
            pl.BlockSpec((BR, 1), lambda i, j: (i, 0)),
            pl.BlockSpec((1, D_HID), lambda i, j: (0, 0)),
            pl.BlockSpec((1, D_HID), lambda i, j: (0, 0)),
            pl.BlockSpec((1, D_HID), lambda i, j: (0, 0)),
            pl.BlockSpec((2, D_HID), lambda i, j: (0, 0)),
            pl.BlockSpec((BR, D_HID), lambda i, j: (i, 0)),
            pl.BlockSpec((D_HID, DH), lambda i, j: (0, j)),
        ],
        out_specs=[
            pl.BlockSpec((BR, D_HID), lambda i, j: (i, 0)),
            pl.BlockSpec((1, BR, DH), lambda i, j: (j, i, 0)),
        ],
        out_shape=[
            jax.ShapeDtypeStruct((N, D_HID), jnp.float32),
            jax.ShapeDtypeStruct((NC, NP, DH), jnp.float32),
        ],
    )(a, a, dis_col, b.reshape(1, D_HID), gamma.reshape(1, D_HID),
      beta.reshape(1, D_HID), stats, h_old, W)


def _tc_finish_readout(a, dis_col, b, gamma, beta, stats, h_old, W_out, b_out):
    dout = W_out.shape[1]

    def body(a0_ref, a1_ref, d_ref, b_ref, g_ref, be_ref, st_ref, hp_ref,
             w_ref, bo_ref, o_ref):
        hh = _bn_silu_res(a0_ref[0], a1_ref[0], d_ref[...], b_ref[...],
                          g_ref[...], be_ref[...], st_ref[...], hp_ref[...])
        o_ref[...] = (
            jnp.dot(hh, w_ref[...], preferred_element_type=jnp.float32)
            + bo_ref[...]
        )

    return pl.pallas_call(
        body,
        grid=(N // BR,),
        in_specs=[
            pl.BlockSpec((1, BR, DH), lambda i: (0, i, 0)),
            pl.BlockSpec((1, BR, DH), lambda i: (1, i, 0)),
            pl.BlockSpec((BR, 1), lambda i: (i, 0)),
            pl.BlockSpec((1, D_HID), lambda i: (0, 0)),
            pl.BlockSpec((1, D_HID), lambda i: (0, 0)),
            pl.BlockSpec((1, D_HID), lambda i: (0, 0)),
            pl.BlockSpec((2, D_HID), lambda i: (0, 0)),
            pl.BlockSpec((BR, D_HID), lambda i: (i, 0)),
            pl.BlockSpec((D_HID, dout), lambda i: (0, 0)),
            pl.BlockSpec((1, dout), lambda i: (0, 0)),
        ],
        out_specs=pl.BlockSpec((BR, dout), lambda i: (i, 0)),
        out_shape=jax.ShapeDtypeStruct((N, dout), jnp.float32),
    )(a, a, dis_col, b.reshape(1, D_HID), gamma.reshape(1, D_HID),
      beta.reshape(1, D_HID), stats, h_old, W_out, b_out.reshape(1, dout))


def kernel(x, edge_index, W_in, b_in, Ws, bs, gammas, betas, W_out, b_out):
    src = edge_index[0].astype(jnp.int32)
    dst = edge_index[1].astype(jnp.int32)

    n_pad = NC * NS * DEG_CH * LANES - E
    dst_deg = jnp.concatenate(
        [dst, jnp.full((n_pad,), NP - 1, jnp.int32)]
    ).reshape(NC, NS, DEG_CH, LANES)

    p_pad = NS * PROP_CH * LANES - E
    src_p = jnp.concatenate([src, jnp.zeros((p_pad,), jnp.int32)])
    dst_p = jnp.concatenate(
        [dst, jnp.full((p_pad,), NP - 1, jnp.int32)]
    ).reshape(NS, PROP_CH, LANES)
    src_prop = jnp.stack([src_p, src_p + NP]).reshape(NC, NS, PROP_CH, LANES)

    deg_part = _sc_degree(dst_deg)
    dis_row = _tc_dis(deg_part)
    dis_col = dis_row[0, :N].reshape(N, 1)

    u = _tc_umat(x, W_in, dis_col)
    a = _sc_propagate(u.reshape(NC * NP, DH), src_prop, dst_p).reshape(NC, NP, DH)
    h, u = _tc_postin_umat(a, dis_col, b_in, Ws[0])
    for l in range(Ws.shape[0]):
        a = _sc_propagate(u.reshape(NC * NP, DH), src_prop, dst_p).reshape(NC, NP, DH)
        st = _tc_stats(a, dis_col, bs[l])
        if l + 1 < Ws.shape[0]:
            h, u = _tc_finish_umat(a, dis_col, bs[l], gammas[l], betas[l],
                                   st, h, Ws[l + 1])
        else:
            return _tc_finish_readout(a, dis_col, bs[l], gammas[l], betas[l],
                                      st, h, W_out, b_out)
